# baseline (speedup 1.0000x reference)
import functools

import jax
import jax.numpy as jnp
from jax import lax
from jax.experimental import pallas as pl
from jax.experimental.pallas import tpu as pltpu
from jax.experimental.pallas import tpu_sc as plsc

N = 10000
NPAD = 10240
D = 128
H = 64
E_NET = 320000
E_DAG = 160000
CH = 128
MQ = 2
SB = 16
NC_NET = -(-E_NET // CH + 7) // 8 * 8
NC_DAG = -(-E_DAG // CH + 7) // 8 * 8
ROWS_PER_TILE = NPAD // 16


def _ceil_div(a, b):
    return (a + b - 1) // b


def _chunk_range(wid, nworkers, nchunks):
    ngroups = nchunks // 8
    base = ngroups // nworkers
    rem = ngroups % nworkers
    gcnt = base + jnp.where(wid < rem, 1, 0)
    gstart = wid * base + jnp.minimum(wid, rem)
    start = pl.multiple_of(gstart * 8, 8)
    cnt = gcnt * 8
    nsb = _ceil_div((base + 1) * 8, SB)
    return start, cnt, nsb


def _deg_body(col_net, ew_net, col_dag, ew_dag, deg_net_out, deg_dag_out,
              deg_net_sp, deg_dag_sp, zbuf, colv, ewv, sem):
    cid = lax.axis_index("c")
    sid = lax.axis_index("s")
    wid = sid * 2 + cid

    for i in range(ROWS_PER_TILE // 16):
        zbuf[pl.ds(i * 16, 16)] = jnp.zeros((16,), jnp.float32)
    pltpu.sync_copy(zbuf, deg_net_sp.at[pl.ds(sid * ROWS_PER_TILE, ROWS_PER_TILE)])
    pltpu.sync_copy(zbuf, deg_dag_sp.at[pl.ds(sid * ROWS_PER_TILE, ROWS_PER_TILE)])
    plsc.subcore_barrier()

    def run_graph(col_hbm, ew_hbm, deg_sp, nchunks):
        start, cnt, nsb = _chunk_range(wid, 32, nchunks)

        def sb_iter(sb, _):
            sb_start = pl.multiple_of(start + sb * SB, 8)
            sb_cnt = jnp.clip(cnt - sb * SB, 0, SB)

            @pl.when(sb_cnt > 0)
            def _():
                pltpu.sync_copy(col_hbm.at[pl.ds(sb_start, SB)], colv)
                pltpu.sync_copy(ew_hbm.at[pl.ds(sb_start, SB)], ewv)

                def ch_iter(j, _):
                    pltpu.async_copy(ewv.at[j], deg_sp.at[colv.at[j]],
                                     sem, add=True)
                    return 0
                lax.fori_loop(0, sb_cnt, ch_iter, 0)

                def drain(j, _):
                    pltpu.make_async_copy(ew_hbm.at[0], ewv.at[0], sem).wait()
                    return 0
                lax.fori_loop(0, sb_cnt, drain, 0)
            return 0

        lax.fori_loop(0, nsb, sb_iter, 0)

    run_graph(col_net, ew_net, deg_net_sp, NC_NET)
    run_graph(col_dag, ew_dag, deg_dag_sp, NC_DAG)
    plsc.subcore_barrier()

    r0 = sid * ROWS_PER_TILE
    pltpu.sync_copy(deg_net_sp.at[pl.ds(r0, ROWS_PER_TILE)],
                    deg_net_out.at[pl.ds(cid * NPAD + r0, ROWS_PER_TILE)])
    pltpu.sync_copy(deg_dag_sp.at[pl.ds(r0, ROWS_PER_TILE)],
                    deg_dag_out.at[pl.ds(cid * NPAD + r0, ROWS_PER_TILE)])


def _deg_call(col_net, ew_net, col_dag, ew_dag):
    return pl.kernel(
        _deg_body,
        out_type=[jax.ShapeDtypeStruct((2 * NPAD,), jnp.float32),
                  jax.ShapeDtypeStruct((2 * NPAD,), jnp.float32)],
        mesh=plsc.VectorSubcoreMesh(core_axis_name="c", subcore_axis_name="s"),
        scratch_types=[
            pltpu.VMEM_SHARED((NPAD,), jnp.float32),
            pltpu.VMEM_SHARED((NPAD,), jnp.float32),
            pltpu.VMEM((ROWS_PER_TILE,), jnp.float32),
            pltpu.VMEM((SB, CH), jnp.int32),
            pltpu.VMEM((SB, CH), jnp.float32),
            pltpu.SemaphoreType.DMA,
        ],
    )(col_net, ew_net, col_dag, ew_dag)


def _mid_body(feat_ref, w_ref, degp_ref, xws_ref, dinv_ref):
    dp = degp_ref[...]
    deg = dp[:, 0:1] + dp[:, 1:2] + 1.0
    dinv = jnp.where(deg > 0, lax.rsqrt(deg), 0.0)
    xw = jnp.dot(feat_ref[...], w_ref[...],
                 preferred_element_type=jnp.float32)
    xws_ref[...] = xw * dinv
    dinv_ref[...] = dinv


def _mid_call(feat_pad, W, degp_t):
    RB = 1024
    grid = (NPAD // RB,)
    return pl.pallas_call(
        _mid_body,
        grid=grid,
        in_specs=[
            pl.BlockSpec((RB, D), lambda i: (i, 0)),
            pl.BlockSpec((D, D), lambda i: (0, 0)),
            pl.BlockSpec((RB, 2), lambda i: (i, 0)),
        ],
        out_specs=[
            pl.BlockSpec((RB, D), lambda i: (i, 0)),
            pl.BlockSpec((RB, 1), lambda i: (i, 0)),
        ],
        out_shape=[jax.ShapeDtypeStruct((NPAD, D), jnp.float32),
                   jax.ShapeDtypeStruct((NPAD, 1), jnp.float32)],
    )(feat_pad, W, degp_t)


def _edge_loop(tbl, row_hbm, col_hbm, ew_hbm, acc_sp, rowv, colv, ewv, msg,
               gsems, ssems, wid, nchunks):
    start, cnt, nsb = _chunk_range(wid, 32, nchunks)

    def sb_iter(sb, _):
        sb_start = pl.multiple_of(start + sb * SB, 8)
        sb_cnt = jnp.clip(cnt - sb * SB, 0, SB)

        @pl.when(sb_cnt > 0)
        def _():
            pltpu.sync_copy(row_hbm.at[pl.ds(sb_start, SB)], rowv)
            pltpu.sync_copy(col_hbm.at[pl.ds(sb_start, SB)], colv)
            pltpu.sync_copy(ew_hbm.at[pl.ds(sb_start, SB)], ewv)

            def scale_buf(b, j):
                def row_iter(g, _):
                    ew16 = ewv[j, pl.ds(g * 16, 16)]
                    for l in range(16):
                        e = g * 16 + l
                        s = ew16[l]
                        for k in range(D // 16):
                            v = msg[b, e, pl.ds(k * 16, 16)]
                            msg[b, e, pl.ds(k * 16, 16)] = v * s
                    return 0
                lax.fori_loop(0, CH // 16, row_iter, 0)

            def gather(j, b):
                return pltpu.async_copy(tbl.at[rowv.at[j]], msg.at[b],
                                        gsems[b])

            def scatter(j, b):
                return pltpu.async_copy(msg.at[b], acc_sp.at[colv.at[j]],
                                        ssems[b], add=True)

            def quad_iter(q, _):
                j0 = q * MQ
                gd = [gather(j0 + u, u) for u in range(MQ)]
                sd = []
                for u in range(MQ):
                    gd[u].wait()
                    scale_buf(u, j0 + u)
                    sd.append(scatter(j0 + u, u))
                for u in range(MQ):
                    sd[u].wait()
                return 0
            lax.fori_loop(0, sb_cnt // MQ, quad_iter, 0)
        return 0

    lax.fori_loop(0, nsb, sb_iter, 0)


def _main_body(xws_net, xws_dag, row_net, col_net, ew_net,
               row_dag, col_dag, ew_dag, acc_net_out, acc_dag_out,
               acc_sp, rowv, colv, ewv, msg, gsem, ssem):
    gsems = [gsem] * MQ
    ssems = [ssem] * MQ
    cid = lax.axis_index("c")
    sid = lax.axis_index("s")
    wid = sid * 2 + cid
    r0 = pl.multiple_of(sid * ROWS_PER_TILE, 8)

    def zero_acc():
        for i in range(ROWS_PER_TILE // CH):
            pltpu.sync_copy(msg.at[0],
                            acc_sp.at[pl.ds(sid * ROWS_PER_TILE + i * CH, CH)])

    def zrow(e, _):
        for k in range(D // 16):
            msg[0, e, pl.ds(k * 16, 16)] = jnp.zeros((16,), jnp.float32)
        return 0
    lax.fori_loop(0, CH, zrow, 0)

    zero_acc()
    plsc.subcore_barrier()
    _edge_loop(xws_net, row_net, col_net, ew_net, acc_sp,
               rowv, colv, ewv, msg, gsems, ssems, wid, NC_NET)
    plsc.subcore_barrier()
    pltpu.sync_copy(acc_sp.at[pl.ds(r0, ROWS_PER_TILE)],
                    acc_net_out.at[cid, pl.ds(r0, ROWS_PER_TILE), :])

    lax.fori_loop(0, CH, zrow, 0)
    zero_acc()
    plsc.subcore_barrier()
    _edge_loop(xws_dag, row_dag, col_dag, ew_dag, acc_sp,
               rowv, colv, ewv, msg, gsems, ssems, wid, NC_DAG)
    plsc.subcore_barrier()
    pltpu.sync_copy(acc_sp.at[pl.ds(r0, ROWS_PER_TILE)],
                    acc_dag_out.at[cid, pl.ds(r0, ROWS_PER_TILE), :])


def _main_call(xws_net, xws_dag, row_net, col_net, ew_net,
               row_dag, col_dag, ew_dag):
    return pl.kernel(
        _main_body,
        out_type=[jax.ShapeDtypeStruct((2, NPAD, D), jnp.float32),
                  jax.ShapeDtypeStruct((2, NPAD, D), jnp.float32)],
        mesh=plsc.VectorSubcoreMesh(core_axis_name="c", subcore_axis_name="s"),
        scratch_types=[
            pltpu.VMEM_SHARED((NPAD, D), jnp.float32),
            pltpu.VMEM((SB, CH), jnp.int32),
            pltpu.VMEM((SB, CH), jnp.int32),
            pltpu.VMEM((SB, CH), jnp.float32),
            pltpu.VMEM((MQ, CH, D), jnp.float32),
            pltpu.SemaphoreType.DMA,
            pltpu.SemaphoreType.DMA,
        ],
    )(xws_net, xws_dag, row_net, col_net, ew_net, row_dag, col_dag, ew_dag)


def _post_body(a0_ref, a1_ref, x_ref, dinv_ref, b_ref, out_ref):
    i = pl.program_id(0)
    a = a0_ref[0] + a1_ref[0]
    t = dinv_ref[...] * (a + x_ref[...]) + b_ref[...]
    r = jax.nn.relu(t)
    rows = i * 1024 + lax.broadcasted_iota(jnp.int32, (1024, 1), 0)
    r = jnp.where(rows < N, r, 0.0)
    s = jnp.sum(r, axis=0, keepdims=True)

    @pl.when(i == 0)
    def _():
        out_ref[...] = jnp.zeros_like(out_ref)

    out_ref[...] += s


def _post_call(acc, xws, dinv, b):
    RB = 1024
    nb = NPAD // RB
    return pl.pallas_call(
        _post_body,
        grid=(nb,),
        in_specs=[
            pl.BlockSpec((1, RB, D), lambda i: (0, i, 0)),
            pl.BlockSpec((1, RB, D), lambda i: (1, i, 0)),
            pl.BlockSpec((RB, D), lambda i: (i, 0)),
            pl.BlockSpec((RB, 1), lambda i: (i, 0)),
            pl.BlockSpec((1, D), lambda i: (0, 0)),
        ],
        out_specs=pl.BlockSpec((1, D), lambda i: (0, 0)),
        out_shape=jax.ShapeDtypeStruct((1, D), jnp.float32),
    )(acc, acc, xws, dinv, b)


def _head_body(a_ref, a1_ref, b1_ref, a2_ref, b2_ref, sn_ref, sd_ref,
               f1_ref, fb1_ref, f2_ref, fb2_ref, out_ref):
    h = jnp.dot(a_ref[...], a1_ref[...], preferred_element_type=jnp.float32)
    h = h + b1_ref[...]
    sp = jnp.maximum(h, 0.0) + jnp.log(1.0 + jnp.exp(-jnp.abs(h)))
    h = h * jnp.tanh(sp)
    act = jnp.dot(h, a2_ref[...], preferred_element_type=jnp.float32) + b2_ref[...]
    mn = sn_ref[...] * (1.0 / N)
    md = sd_ref[...] * (1.0 / N)
    c = jnp.concatenate([mn, md, act], axis=1)
    h2 = jnp.dot(c, f1_ref[...], preferred_element_type=jnp.float32) + fb1_ref[...]
    h2 = jax.nn.relu(h2)
    out_ref[...] = jnp.dot(h2, f2_ref[...],
                           preferred_element_type=jnp.float32) + fb2_ref[...]


def _head_call(action, A1, b1, A2, b2, sum_net, sum_dag, F1, fb1, F2, fb2):
    return pl.pallas_call(
        _head_body,
        out_shape=jax.ShapeDtypeStruct((1, 1), jnp.float32),
    )(action.reshape(1, -1), A1, b1.reshape(1, -1), A2, b2.reshape(1, -1),
      sum_net, sum_dag, F1, fb1.reshape(1, -1), F2, fb2.reshape(1, 1))


def kernel(net_feat, net_edge_index, net_edge_weights, dag_feat, dag_edge_index,
           dag_edge_weights, action, net_W, net_b, dag_W, dag_b, A1, b1, A2, b2,
           F1, fb1, F2, fb2):
    def prep(arr, nc):
        flat = arr.reshape(-1)
        return jnp.pad(flat, (0, (nc + SB) * CH - flat.shape[0])).reshape(
            nc + SB, CH)

    row_net = prep(net_edge_index[0], NC_NET)
    col_net = prep(net_edge_index[1], NC_NET)
    ew_net = prep(net_edge_weights, NC_NET)
    row_dag = prep(dag_edge_index[0], NC_DAG)
    col_dag = prep(dag_edge_index[1], NC_DAG)
    ew_dag = prep(dag_edge_weights, NC_DAG)

    feat_net = jnp.pad(net_feat, ((0, NPAD - N), (0, 0)))
    feat_dag = jnp.pad(dag_feat, ((0, NPAD - N), (0, 0)))

    degp_net, degp_dag = _deg_call(col_net, ew_net, col_dag, ew_dag)

    xws_net, dinv_net = _mid_call(feat_net, net_W, degp_net.reshape(2, NPAD).T)
    xws_dag, dinv_dag = _mid_call(feat_dag, dag_W, degp_dag.reshape(2, NPAD).T)

    acc_net, acc_dag = _main_call(xws_net, xws_dag, row_net, col_net, ew_net,
                                  row_dag, col_dag, ew_dag)

    sum_net = _post_call(acc_net, xws_net, dinv_net, net_b.reshape(1, -1))
    sum_dag = _post_call(acc_dag, xws_dag, dinv_dag, dag_b.reshape(1, -1))

    out = _head_call(action, A1, b1, A2, b2, sum_net, sum_dag, F1, fb1, F2, fb2)
    return out.reshape((1,))


# speedup vs baseline: 1.0038x; 1.0038x over previous
import functools

import jax
import jax.numpy as jnp
from jax import lax
from jax.experimental import pallas as pl
from jax.experimental.pallas import tpu as pltpu
from jax.experimental.pallas import tpu_sc as plsc

N = 10000
NPAD = 10240
D = 128
H = 64
E_NET = 320000
E_DAG = 160000
CH = 128
MQ = 2
SB = 16
NC_NET = -(-E_NET // CH + 7) // 8 * 8
NC_DAG = -(-E_DAG // CH + 7) // 8 * 8
ROWS_PER_TILE = NPAD // 16


def _ceil_div(a, b):
    return (a + b - 1) // b


def _chunk_range(wid, nworkers, nchunks):
    ngroups = nchunks // 8
    base = ngroups // nworkers
    rem = ngroups % nworkers
    gcnt = base + jnp.where(wid < rem, 1, 0)
    gstart = wid * base + jnp.minimum(wid, rem)
    start = pl.multiple_of(gstart * 8, 8)
    cnt = gcnt * 8
    nsb = _ceil_div((base + 1) * 8, SB)
    return start, cnt, nsb


def _deg_body(col_net, ew_net, col_dag, ew_dag, deg_net_out, deg_dag_out,
              deg_net_sp, deg_dag_sp, zbuf, colv, ewv, sem):
    cid = lax.axis_index("c")
    sid = lax.axis_index("s")
    wid = sid * 2 + cid

    for i in range(ROWS_PER_TILE // 16):
        zbuf[pl.ds(i * 16, 16)] = jnp.zeros((16,), jnp.float32)
    pltpu.sync_copy(zbuf, deg_net_sp.at[pl.ds(sid * ROWS_PER_TILE, ROWS_PER_TILE)])
    pltpu.sync_copy(zbuf, deg_dag_sp.at[pl.ds(sid * ROWS_PER_TILE, ROWS_PER_TILE)])
    plsc.subcore_barrier()

    def run_graph(col_hbm, ew_hbm, deg_sp, nchunks):
        start, cnt, nsb = _chunk_range(wid, 32, nchunks)

        def sb_iter(sb, _):
            sb_start = pl.multiple_of(start + sb * SB, 8)
            sb_cnt = jnp.clip(cnt - sb * SB, 0, SB)

            @pl.when(sb_cnt > 0)
            def _():
                pltpu.sync_copy(col_hbm.at[pl.ds(sb_start, SB)], colv)
                pltpu.sync_copy(ew_hbm.at[pl.ds(sb_start, SB)], ewv)

                def ch_iter(j, _):
                    pltpu.async_copy(ewv.at[j], deg_sp.at[colv.at[j]],
                                     sem, add=True)
                    return 0
                lax.fori_loop(0, sb_cnt, ch_iter, 0)

                def drain(j, _):
                    pltpu.make_async_copy(ew_hbm.at[0], ewv.at[0], sem).wait()
                    return 0
                lax.fori_loop(0, sb_cnt, drain, 0)
            return 0

        lax.fori_loop(0, nsb, sb_iter, 0)

    run_graph(col_net, ew_net, deg_net_sp, NC_NET)
    run_graph(col_dag, ew_dag, deg_dag_sp, NC_DAG)
    plsc.subcore_barrier()

    r0 = sid * ROWS_PER_TILE
    pltpu.sync_copy(deg_net_sp.at[pl.ds(r0, ROWS_PER_TILE)],
                    deg_net_out.at[pl.ds(cid * NPAD + r0, ROWS_PER_TILE)])
    pltpu.sync_copy(deg_dag_sp.at[pl.ds(r0, ROWS_PER_TILE)],
                    deg_dag_out.at[pl.ds(cid * NPAD + r0, ROWS_PER_TILE)])


def _deg_call(col_net, ew_net, col_dag, ew_dag):
    return pl.kernel(
        _deg_body,
        out_type=[jax.ShapeDtypeStruct((2 * NPAD,), jnp.float32),
                  jax.ShapeDtypeStruct((2 * NPAD,), jnp.float32)],
        mesh=plsc.VectorSubcoreMesh(core_axis_name="c", subcore_axis_name="s"),
        scratch_types=[
            pltpu.VMEM_SHARED((NPAD,), jnp.float32),
            pltpu.VMEM_SHARED((NPAD,), jnp.float32),
            pltpu.VMEM((ROWS_PER_TILE,), jnp.float32),
            pltpu.VMEM((SB, CH), jnp.int32),
            pltpu.VMEM((SB, CH), jnp.float32),
            pltpu.SemaphoreType.DMA,
        ],
    )(col_net, ew_net, col_dag, ew_dag)


def _mid_body(feat_ref, w_ref, degp_ref, xws_ref, dinv_ref):
    dp = degp_ref[0]
    deg = dp[:, 0:1] + dp[:, 1:2] + 1.0
    dinv = jnp.where(deg > 0, lax.rsqrt(deg), 0.0)
    xw = jnp.dot(feat_ref[0], w_ref[0],
                 preferred_element_type=jnp.float32)
    xws_ref[...] = (xw * dinv)[None]
    dinv_ref[...] = dinv[None]


def _mid_call(feats2, W2, degp2):
    RB = 1024
    return pl.pallas_call(
        _mid_body,
        grid=(2, NPAD // RB),
        in_specs=[
            pl.BlockSpec((1, RB, D), lambda g, i: (g, i, 0)),
            pl.BlockSpec((1, D, D), lambda g, i: (g, 0, 0)),
            pl.BlockSpec((1, RB, 2), lambda g, i: (g, i, 0)),
        ],
        out_specs=[
            pl.BlockSpec((1, RB, D), lambda g, i: (g, i, 0)),
            pl.BlockSpec((1, RB, 1), lambda g, i: (g, i, 0)),
        ],
        out_shape=[jax.ShapeDtypeStruct((2, NPAD, D), jnp.float32),
                   jax.ShapeDtypeStruct((2, NPAD, 1), jnp.float32)],
    )(feats2, W2, degp2)


def _edge_loop(tbl, row_hbm, col_hbm, ew_hbm, acc_sp, rowv, colv, ewv, msg,
               gsems, ssems, wid, nchunks):
    start, cnt, nsb = _chunk_range(wid, 32, nchunks)

    def sb_iter(sb, _):
        sb_start = pl.multiple_of(start + sb * SB, 8)
        sb_cnt = jnp.clip(cnt - sb * SB, 0, SB)

        @pl.when(sb_cnt > 0)
        def _():
            pltpu.sync_copy(row_hbm.at[pl.ds(sb_start, SB)], rowv)
            pltpu.sync_copy(col_hbm.at[pl.ds(sb_start, SB)], colv)
            pltpu.sync_copy(ew_hbm.at[pl.ds(sb_start, SB)], ewv)

            def scale_buf(b, j):
                def row_iter(g, _):
                    ew16 = ewv[j, pl.ds(g * 16, 16)]
                    for l in range(16):
                        e = g * 16 + l
                        s = ew16[l]
                        for k in range(D // 16):
                            v = msg[b, e, pl.ds(k * 16, 16)]
                            msg[b, e, pl.ds(k * 16, 16)] = v * s
                    return 0
                lax.fori_loop(0, CH // 16, row_iter, 0)

            def gather(j, b):
                return pltpu.async_copy(tbl.at[rowv.at[j]], msg.at[b],
                                        gsems[b])

            def scatter(j, b):
                return pltpu.async_copy(msg.at[b], acc_sp.at[colv.at[j]],
                                        ssems[b], add=True)

            def quad_iter(q, _):
                j0 = q * MQ
                gd = [gather(j0 + u, u) for u in range(MQ)]
                sd = []
                for u in range(MQ):
                    gd[u].wait()
                    scale_buf(u, j0 + u)
                    sd.append(scatter(j0 + u, u))
                for u in range(MQ):
                    sd[u].wait()
                return 0
            lax.fori_loop(0, sb_cnt // MQ, quad_iter, 0)
        return 0

    lax.fori_loop(0, nsb, sb_iter, 0)


def _main_body(xws2, row_net, col_net, ew_net,
               row_dag, col_dag, ew_dag, accs_out,
               acc_sp, rowv, colv, ewv, msg, gsem, ssem):
    gsems = [gsem] * MQ
    ssems = [ssem] * MQ
    cid = lax.axis_index("c")
    sid = lax.axis_index("s")
    wid = sid * 2 + cid
    r0 = pl.multiple_of(sid * ROWS_PER_TILE, 8)

    def zero_acc():
        for i in range(ROWS_PER_TILE // CH):
            pltpu.sync_copy(msg.at[0],
                            acc_sp.at[pl.ds(sid * ROWS_PER_TILE + i * CH, CH)])

    def zrow(e, _):
        for k in range(D // 16):
            msg[0, e, pl.ds(k * 16, 16)] = jnp.zeros((16,), jnp.float32)
        return 0
    lax.fori_loop(0, CH, zrow, 0)

    zero_acc()
    plsc.subcore_barrier()
    _edge_loop(xws2.at[0], row_net, col_net, ew_net, acc_sp,
               rowv, colv, ewv, msg, gsems, ssems, wid, NC_NET)
    plsc.subcore_barrier()
    pltpu.sync_copy(acc_sp.at[pl.ds(r0, ROWS_PER_TILE)],
                    accs_out.at[0, cid, pl.ds(r0, ROWS_PER_TILE), :])

    lax.fori_loop(0, CH, zrow, 0)
    zero_acc()
    plsc.subcore_barrier()
    _edge_loop(xws2.at[1], row_dag, col_dag, ew_dag, acc_sp,
               rowv, colv, ewv, msg, gsems, ssems, wid, NC_DAG)
    plsc.subcore_barrier()
    pltpu.sync_copy(acc_sp.at[pl.ds(r0, ROWS_PER_TILE)],
                    accs_out.at[1, cid, pl.ds(r0, ROWS_PER_TILE), :])


def _main_call(xws2, row_net, col_net, ew_net,
               row_dag, col_dag, ew_dag):
    return pl.kernel(
        _main_body,
        out_type=jax.ShapeDtypeStruct((2, 2, NPAD, D), jnp.float32),
        mesh=plsc.VectorSubcoreMesh(core_axis_name="c", subcore_axis_name="s"),
        scratch_types=[
            pltpu.VMEM_SHARED((NPAD, D), jnp.float32),
            pltpu.VMEM((SB, CH), jnp.int32),
            pltpu.VMEM((SB, CH), jnp.int32),
            pltpu.VMEM((SB, CH), jnp.float32),
            pltpu.VMEM((MQ, CH, D), jnp.float32),
            pltpu.SemaphoreType.DMA,
            pltpu.SemaphoreType.DMA,
        ],
    )(xws2, row_net, col_net, ew_net, row_dag, col_dag, ew_dag)


def _post_body(a0_ref, a1_ref, x_ref, dinv_ref, b_ref, out_ref):
    i = pl.program_id(1)
    a = a0_ref[0, 0] + a1_ref[0, 0]
    t = dinv_ref[0] * (a + x_ref[0]) + b_ref[0]
    r = jax.nn.relu(t)
    rows = i * 1024 + lax.broadcasted_iota(jnp.int32, (1024, 1), 0)
    r = jnp.where(rows < N, r, 0.0)
    s = jnp.sum(r, axis=0, keepdims=True)

    @pl.when(i == 0)
    def _():
        out_ref[...] = jnp.zeros_like(out_ref)

    out_ref[...] += s[None]


def _post_call(accs, xws2, dinv2, b2):
    RB = 1024
    return pl.pallas_call(
        _post_body,
        grid=(2, NPAD // RB),
        in_specs=[
            pl.BlockSpec((1, 1, RB, D), lambda g, i: (g, 0, i, 0)),
            pl.BlockSpec((1, 1, RB, D), lambda g, i: (g, 1, i, 0)),
            pl.BlockSpec((1, RB, D), lambda g, i: (g, i, 0)),
            pl.BlockSpec((1, RB, 1), lambda g, i: (g, i, 0)),
            pl.BlockSpec((1, 1, D), lambda g, i: (g, 0, 0)),
        ],
        out_specs=pl.BlockSpec((1, 1, D), lambda g, i: (g, 0, 0)),
        out_shape=jax.ShapeDtypeStruct((2, 1, D), jnp.float32),
    )(accs, accs, xws2, dinv2, b2)


def _head_body(a_ref, a1_ref, b1_ref, a2_ref, b2_ref, sn_ref, sd_ref,
               f1_ref, fb1_ref, f2_ref, fb2_ref, out_ref):
    h = jnp.dot(a_ref[...], a1_ref[...], preferred_element_type=jnp.float32)
    h = h + b1_ref[...]
    sp = jnp.maximum(h, 0.0) + jnp.log(1.0 + jnp.exp(-jnp.abs(h)))
    h = h * jnp.tanh(sp)
    act = jnp.dot(h, a2_ref[...], preferred_element_type=jnp.float32) + b2_ref[...]
    mn = sn_ref[...] * (1.0 / N)
    md = sd_ref[...] * (1.0 / N)
    c = jnp.concatenate([mn, md, act], axis=1)
    h2 = jnp.dot(c, f1_ref[...], preferred_element_type=jnp.float32) + fb1_ref[...]
    h2 = jax.nn.relu(h2)
    out_ref[...] = jnp.dot(h2, f2_ref[...],
                           preferred_element_type=jnp.float32) + fb2_ref[...]


def _head_call(action, A1, b1, A2, b2, sum_net, sum_dag, F1, fb1, F2, fb2):
    return pl.pallas_call(
        _head_body,
        out_shape=jax.ShapeDtypeStruct((1, 1), jnp.float32),
    )(action.reshape(1, -1), A1, b1.reshape(1, -1), A2, b2.reshape(1, -1),
      sum_net, sum_dag, F1, fb1.reshape(1, -1), F2, fb2.reshape(1, 1))


def kernel(net_feat, net_edge_index, net_edge_weights, dag_feat, dag_edge_index,
           dag_edge_weights, action, net_W, net_b, dag_W, dag_b, A1, b1, A2, b2,
           F1, fb1, F2, fb2):
    def prep(arr, nc):
        flat = arr.reshape(-1)
        return jnp.pad(flat, (0, (nc + SB) * CH - flat.shape[0])).reshape(
            nc + SB, CH)

    row_net = prep(net_edge_index[0], NC_NET)
    col_net = prep(net_edge_index[1], NC_NET)
    ew_net = prep(net_edge_weights, NC_NET)
    row_dag = prep(dag_edge_index[0], NC_DAG)
    col_dag = prep(dag_edge_index[1], NC_DAG)
    ew_dag = prep(dag_edge_weights, NC_DAG)

    feats2 = jnp.stack([jnp.pad(net_feat, ((0, NPAD - N), (0, 0))),
                        jnp.pad(dag_feat, ((0, NPAD - N), (0, 0)))])
    W2 = jnp.stack([net_W, dag_W])
    b2g = jnp.stack([net_b, dag_b]).reshape(2, 1, D)

    degp_net, degp_dag = _deg_call(col_net, ew_net, col_dag, ew_dag)
    degp2 = jnp.stack([degp_net.reshape(2, NPAD).T,
                       degp_dag.reshape(2, NPAD).T])

    xws2, dinv2 = _mid_call(feats2, W2, degp2)

    accs = _main_call(xws2, row_net, col_net, ew_net,
                      row_dag, col_dag, ew_dag)

    sums = _post_call(accs, xws2, dinv2, b2g)

    out = _head_call(action, A1, b1, A2, b2, sums[0], sums[1],
                     F1, fb1, F2, fb2)
    return out.reshape((1,))
